# baseline (speedup 1.0000x reference)
import functools

import jax
import jax.numpy as jnp
from jax import lax
from jax.experimental import pallas as pl
from jax.experimental.pallas import tpu as pltpu
from jax.experimental.pallas import tpu_sc as plsc

_NUM_EMBEDDINGS = 1024
_EMBEDDING_DIM = 64
_COMMITMENT_COST = 0.25
_TOKENS_TOTAL = 16 * 1024
_BLOCK = 1024

_SC_INFO = plsc.get_sparse_core_info()
_NC = _SC_INFO.num_cores
_NS = _SC_INFO.num_subcores
_NW = _NC * _NS
_BPW = _TOKENS_TOTAL // _NW
_CHUNK = 128
_NCHUNK = _BPW // _CHUNK


def _vq_block(x_ref, cb_ref, idx_ref, sse_ref):
    i = pl.program_id(0)
    x = x_ref[...]
    cb = cb_ref[...]
    xx = jnp.sum(x * x, axis=1, keepdims=True)
    ee = jnp.sum(cb * cb, axis=1)
    m2xe = lax.dot_general(
        x * -2.0, cb, (((1,), (1,)), ((), ())),
        preferred_element_type=jnp.float32,
        precision=lax.Precision.DEFAULT,
    )
    dist = xx + m2xe + ee[None, :]
    idx_ref[...] = jnp.argmin(dist, axis=1)
    part = jnp.sum(jnp.min(dist, axis=1))

    @pl.when(i == 0)
    def _init():
        sse_ref[0, 0] = 0.0

    sse_ref[0, 0] += part


@functools.partial(
    pl.kernel,
    mesh=plsc.VectorSubcoreMesh(core_axis_name="c", subcore_axis_name="s"),
    compiler_params=pltpu.CompilerParams(use_tc_tiling_on_sc=False),
    out_type=jax.ShapeDtypeStruct((_TOKENS_TOTAL, _EMBEDDING_DIM), jnp.float32),
    scratch_types=[
        pltpu.VMEM((_NCHUNK, _CHUNK), jnp.int32),
        pltpu.VMEM((_BPW, _EMBEDDING_DIM), jnp.float32),
        pltpu.SemaphoreType.DMA,
    ],
)
def _sc_gather(table_hbm, idx_hbm, out_hbm, idx_v, rows_v, sem):
    wid = lax.axis_index("s") * _NC + lax.axis_index("c")
    pltpu.sync_copy(idx_hbm.at[wid], idx_v)
    copies = [
        pltpu.async_copy(
            table_hbm.at[idx_v.at[k]],
            rows_v.at[pl.ds(k * _CHUNK, _CHUNK)],
            sem,
        )
        for k in range(_NCHUNK)
    ]
    for c in copies:
        c.wait()
    pltpu.sync_copy(rows_v, out_hbm.at[pl.ds(wid * _BPW, _BPW)])


def kernel(inputs, codebook):
    flat = inputs.reshape(-1, _EMBEDDING_DIM)
    n_tokens = flat.shape[0]
    grid = n_tokens // _BLOCK
    idx, sse = pl.pallas_call(
        _vq_block,
        grid=(grid,),
        in_specs=[
            pl.BlockSpec((_BLOCK, _EMBEDDING_DIM), lambda i: (i, 0)),
            pl.BlockSpec((_NUM_EMBEDDINGS, _EMBEDDING_DIM), lambda i: (0, 0)),
        ],
        out_specs=[
            pl.BlockSpec((_BLOCK,), lambda i: (i,)),
            pl.BlockSpec(memory_space=pltpu.SMEM, block_shape=(1, 1),
                         index_map=lambda i: (0, 0)),
        ],
        out_shape=[
            jax.ShapeDtypeStruct((n_tokens,), jnp.int32),
            jax.ShapeDtypeStruct((1, 1), jnp.float32),
        ],
    )(flat, codebook)
    q = _sc_gather(codebook, idx.reshape(_NW, _NCHUNK, _CHUNK))
    loss = sse[0, 0] * (_COMMITMENT_COST / flat.size)
    return (loss, q.reshape(inputs.shape))


# speedup vs baseline: 1.3090x; 1.3090x over previous
import jax
import jax.numpy as jnp
from jax import lax
from jax.experimental import pallas as pl
from jax.experimental.pallas import tpu as pltpu

_NUM_EMBEDDINGS = 1024
_EMBEDDING_DIM = 64
_COMMITMENT_COST = 0.25
_BLOCK = 1024


def _vq_block(x_ref, cb_ref, rhs_ref, q_ref, sse_ref):
    i = pl.program_id(0)
    x = x_ref[...]
    cb = cb_ref[...]
    xx = jnp.sum(x * x, axis=1, keepdims=True)
    ee = jnp.sum(cb * cb, axis=1)
    m2xe = lax.dot_general(
        x * -2.0, cb, (((1,), (1,)), ((), ())),
        preferred_element_type=jnp.float32,
        precision=lax.Precision.DEFAULT,
    )
    dist = xx + m2xe + ee[None, :]
    minv = jnp.min(dist, axis=1, keepdims=True)
    maskf = jnp.where(dist == minv, 1.0, 0.0)
    ext = lax.dot_general(
        maskf, rhs_ref[...], (((1,), (0,)), ((), ())),
        preferred_element_type=jnp.float32,
        precision=lax.Precision.DEFAULT,
    )
    q_ref[...] = ext[:, :_EMBEDDING_DIM]
    ties = jnp.max(ext[:, _EMBEDDING_DIM]) > 1.5

    @pl.when(ties)
    def _fix():
        iota = lax.broadcasted_iota(jnp.int32, dist.shape, 1)
        idx = jnp.min(jnp.where(dist == minv, iota, _NUM_EMBEDDINGS), axis=1)
        onehot = jnp.where(iota == idx[:, None], 1.0, 0.0)
        q_ref[...] = lax.dot_general(
            onehot, cb, (((1,), (0,)), ((), ())),
            preferred_element_type=jnp.float32,
            precision=lax.Precision.DEFAULT,
        )

    part = jnp.sum(minv)

    @pl.when(i == 0)
    def _init():
        sse_ref[0, 0] = 0.0

    sse_ref[0, 0] += part


def kernel(inputs, codebook):
    flat = inputs.reshape(-1, _EMBEDDING_DIM)
    n_tokens = flat.shape[0]
    grid = n_tokens // _BLOCK
    rhs = jnp.concatenate(
        [codebook, jnp.ones((_NUM_EMBEDDINGS, 1), jnp.float32)], axis=1)
    q, sse = pl.pallas_call(
        _vq_block,
        grid=(grid,),
        in_specs=[
            pl.BlockSpec((_BLOCK, _EMBEDDING_DIM), lambda i: (i, 0)),
            pl.BlockSpec((_NUM_EMBEDDINGS, _EMBEDDING_DIM), lambda i: (0, 0)),
            pl.BlockSpec((_NUM_EMBEDDINGS, _EMBEDDING_DIM + 1),
                         lambda i: (0, 0)),
        ],
        out_specs=[
            pl.BlockSpec((_BLOCK, _EMBEDDING_DIM), lambda i: (i, 0)),
            pl.BlockSpec(memory_space=pltpu.SMEM, block_shape=(1, 1),
                         index_map=lambda i: (0, 0)),
        ],
        out_shape=[
            jax.ShapeDtypeStruct((n_tokens, _EMBEDDING_DIM), jnp.float32),
            jax.ShapeDtypeStruct((1, 1), jnp.float32),
        ],
    )(flat, codebook, rhs)
    loss = sse[0, 0] * (_COMMITMENT_COST / flat.size)
    return (loss, q.reshape(inputs.shape))
